# baseline (speedup 1.0000x reference)
import functools

import jax
import jax.numpy as jnp
from jax import lax
from jax.experimental import pallas as pl
from jax.experimental.pallas import tpu as pltpu
from jax.experimental.pallas import tpu_sc as plsc

B = 16384
D = 16
FEAT = 1000000
PACK = 8
ROW128 = FEAT // PACK
NC = 2
NS = 16
L = 16
NW = NC * NS
BPW = B // NW
IPW = 2 * BPW
ICHUNK = 128
NCHUNK = IPW // ICHUNK
EPC = ICHUNK // 2
GPC = EPC // L


def _lfm_body(x_hbm, table_hbm, out_hbm, idx_v, hi_v, buf_a, buf_b, out_v,
              sem_a, sem_b):
    wid = lax.axis_index("s") * NC + lax.axis_index("c")

    pltpu.sync_copy(x_hbm.at[pl.ds(wid * NCHUNK, NCHUNK)], idx_v)
    for t in range(NCHUNK):
        for c in range(ICHUNK // L):
            hi_v[t, pl.ds(c * L, L)] = lax.shift_right_logical(
                idx_v[t, pl.ds(c * L, L)], 3
            )

    bufs = [buf_a, buf_b]
    lanes = lax.iota(jnp.int32, L)

    sems = [sem_a, sem_b]

    def fire(j):
        return pltpu.async_copy(table_hbm.at[hi_v.at[j]], bufs[j % 2], sems[j % 2])

    def compute(j):
        buf = bufs[j % 2]
        jvec = jnp.full((L,), j, jnp.int32)
        for g in range(GPC):
            k0 = 2 * (g * L) + 2 * lanes
            k1 = k0 + 1
            i0 = plsc.load_gather(idx_v, [jvec, k0])
            i1 = plsc.load_gather(idx_v, [jvec, k1])
            c0 = (i0 & 7) * D
            c1 = (i1 & 7) * D
            acc = jnp.zeros((L,), jnp.float32)
            for d in range(D):
                a = plsc.load_gather(buf, [k0, c0 + d])
                b = plsc.load_gather(buf, [k1, c1 + d])
                acc = acc + a * b
            out_v[pl.ds(j * EPC + g * L, L)] = 1.0 / (1.0 + jnp.exp(-acc))

    copies = [fire(0), fire(1)]
    for j in range(NCHUNK):
        copies[j].wait()
        compute(j)
        if j + 2 < NCHUNK:
            copies.append(fire(j + 2))

    pltpu.sync_copy(out_v, out_hbm.at[pl.ds(wid * BPW, BPW)])


@functools.partial(
    pl.kernel,
    out_type=jax.ShapeDtypeStruct((B,), jnp.float32),
    mesh=plsc.VectorSubcoreMesh(core_axis_name="c", subcore_axis_name="s"),
    compiler_params=pltpu.CompilerParams(needs_layout_passes=False),
    scratch_types=[
        pltpu.VMEM((NCHUNK, ICHUNK), jnp.int32),
        pltpu.VMEM((NCHUNK, ICHUNK), jnp.int32),
        pltpu.VMEM((ICHUNK, PACK * D), jnp.float32),
        pltpu.VMEM((ICHUNK, PACK * D), jnp.float32),
        pltpu.VMEM((BPW,), jnp.float32),
        pltpu.SemaphoreType.DMA,
        pltpu.SemaphoreType.DMA,
    ],
)
def _lfm_sc(x_hbm, table_hbm, out_hbm, idx_v, hi_v, buf_a, buf_b, out_v,
            sem_a, sem_b):
    _lfm_body(x_hbm, table_hbm, out_hbm, idx_v, hi_v, buf_a, buf_b, out_v,
              sem_a, sem_b)


def kernel(x, table):
    x2 = x.astype(jnp.int32).reshape(NW * NCHUNK, ICHUNK)
    t128 = table.reshape(ROW128, PACK * D)
    out = _lfm_sc(x2, t128)
    return out.reshape(B, 1)


# speedup vs baseline: 1.0269x; 1.0269x over previous
import functools

import jax
import jax.numpy as jnp
from jax import lax
from jax.experimental import pallas as pl
from jax.experimental.pallas import tpu as pltpu
from jax.experimental.pallas import tpu_sc as plsc

B = 16384
D = 16
FEAT = 1000000
PACK = 8
ROW128 = FEAT // PACK
NC = 2
NS = 16
L = 16
NW = NC * NS
BPW = B // NW
IPW = 2 * BPW
ICHUNK = 128
NCHUNK = IPW // ICHUNK
EPC = ICHUNK // 2
GPC = EPC // L

NCOL = FEAT // ICHUNK
CPW = NCOL // NW
NEXTRA = NCOL - CPW * NW
MAINL = NCOL * L


def _retile_body(tt_hbm, tail_hbm, w2_hbm, buf_a, buf_b, tb_a, tb_b,
                 sin_a, sin_b, sout_a, sout_b):
    wid = lax.axis_index("s") * NC + lax.axis_index("c")
    base = wid * CPW
    lanes = lax.iota(jnp.int32, L)

    bufs = [buf_a, buf_b]
    tbs = [tb_a, tb_b]
    sins = [sin_a, sin_b]
    souts = [sout_a, sout_b]

    def start_in(c, p):
        pltpu.make_async_copy(
            tt_hbm.at[:, pl.ds(c * ICHUNK, ICHUNK)], bufs[p], sins[p]
        ).start()

    def wait_in(p):
        pltpu.make_async_copy(
            tt_hbm.at[:, pl.ds(0, ICHUNK)], bufs[p], sins[p]
        ).wait()

    def start_out(c, p):
        pltpu.make_async_copy(
            tbs[p], w2_hbm.at[pl.ds(c * L, L), :], souts[p]
        ).start()

    def wait_out(p):
        pltpu.make_async_copy(
            tbs[p], w2_hbm.at[pl.ds(0, L), :], souts[p]
        ).wait()

    def transpose_col(p, nl=ICHUNK):
        buf, tb = bufs[p], tbs[p]
        for l in range(nl):
            v = plsc.load_gather(buf, [lanes, jnp.full((L,), l, jnp.int32)])
            tb[l // PACK, pl.ds(D * (l % PACK), D)] = v

    start_in(base, 0)

    def body(j, _):
        c0 = base + 2 * j
        start_in(c0 + 1, 1)
        wait_in(0)
        transpose_col(0)

        @pl.when(j > 0)
        def _():
            wait_out(0)

        start_out(c0, 0)

        @pl.when(j < CPW // 2 - 1)
        def _():
            start_in(c0 + 2, 0)

        wait_in(1)
        transpose_col(1)

        @pl.when(j > 0)
        def _():
            wait_out(1)

        start_out(c0 + 1, 1)
        return 0

    lax.fori_loop(0, CPW // 2, body, 0)
    wait_out(0)
    wait_out(1)

    for e in range(NEXTRA):
        @pl.when(wid == e)
        def _(e=e):
            c = NW * CPW + e
            pltpu.sync_copy(tt_hbm.at[:, pl.ds(c * ICHUNK, ICHUNK)], buf_a)
            transpose_col(0)
            pltpu.sync_copy(tb_a, w2_hbm.at[pl.ds(c * L, L), :])

    @pl.when(wid == NW - 1)
    def _():
        pltpu.sync_copy(tail_hbm, tb_a.at[pl.ds(0, PACK)])
        pltpu.sync_copy(tb_a.at[pl.ds(0, PACK)], w2_hbm.at[pl.ds(MAINL, PACK), :])


@functools.partial(
    pl.kernel,
    out_type=jax.ShapeDtypeStruct((ROW128, PACK * D), jnp.float32),
    mesh=plsc.VectorSubcoreMesh(core_axis_name="c", subcore_axis_name="s"),
    compiler_params=pltpu.CompilerParams(needs_layout_passes=False),
    scratch_types=[
        pltpu.VMEM((D, ICHUNK), jnp.float32),
        pltpu.VMEM((D, ICHUNK), jnp.float32),
        pltpu.VMEM((L, PACK * D), jnp.float32),
        pltpu.VMEM((L, PACK * D), jnp.float32),
        pltpu.SemaphoreType.DMA,
        pltpu.SemaphoreType.DMA,
        pltpu.SemaphoreType.DMA,
        pltpu.SemaphoreType.DMA,
    ],
)
def _retile_sc(tt_hbm, tail_hbm, w2_hbm, buf_a, buf_b, tb_a, tb_b,
               sin_a, sin_b, sout_a, sout_b):
    _retile_body(tt_hbm, tail_hbm, w2_hbm, buf_a, buf_b, tb_a, tb_b,
                 sin_a, sin_b, sout_a, sout_b)


def _lfm_body(x_hbm, table_hbm, out_hbm, idx_v, hi_v, buf_a, buf_b, out_v,
              sem_a, sem_b):
    wid = lax.axis_index("s") * NC + lax.axis_index("c")

    pltpu.sync_copy(x_hbm.at[pl.ds(wid * NCHUNK, NCHUNK)], idx_v)
    for t in range(NCHUNK):
        for c in range(ICHUNK // L):
            hi_v[t, pl.ds(c * L, L)] = lax.shift_right_logical(
                idx_v[t, pl.ds(c * L, L)], 3
            )

    bufs = [buf_a, buf_b]
    sems = [sem_a, sem_b]
    lanes = lax.iota(jnp.int32, L)

    def fire(j):
        return pltpu.async_copy(
            table_hbm.at[hi_v.at[j]], bufs[j % 2], sems[j % 2]
        )

    def compute(j):
        buf = bufs[j % 2]
        jvec = jnp.full((L,), j, jnp.int32)
        for g in range(GPC):
            k0 = 2 * (g * L) + 2 * lanes
            k1 = k0 + 1
            i0 = plsc.load_gather(idx_v, [jvec, k0])
            i1 = plsc.load_gather(idx_v, [jvec, k1])
            c0 = (i0 & 7) * D
            c1 = (i1 & 7) * D
            acc = jnp.zeros((L,), jnp.float32)
            for d in range(D):
                a = plsc.load_gather(buf, [k0, c0 + d])
                b = plsc.load_gather(buf, [k1, c1 + d])
                acc = acc + a * b
            out_v[pl.ds(j * EPC + g * L, L)] = 1.0 / (1.0 + jnp.exp(-acc))

    copies = [fire(0), fire(1)]
    for j in range(NCHUNK):
        copies[j].wait()
        compute(j)
        if j + 2 < NCHUNK:
            copies.append(fire(j + 2))

    pltpu.sync_copy(out_v, out_hbm.at[pl.ds(wid * BPW, BPW)])


@functools.partial(
    pl.kernel,
    out_type=jax.ShapeDtypeStruct((B,), jnp.float32),
    mesh=plsc.VectorSubcoreMesh(core_axis_name="c", subcore_axis_name="s"),
    compiler_params=pltpu.CompilerParams(needs_layout_passes=False),
    scratch_types=[
        pltpu.VMEM((NCHUNK, ICHUNK), jnp.int32),
        pltpu.VMEM((NCHUNK, ICHUNK), jnp.int32),
        pltpu.VMEM((ICHUNK, PACK * D), jnp.float32),
        pltpu.VMEM((ICHUNK, PACK * D), jnp.float32),
        pltpu.VMEM((BPW,), jnp.float32),
        pltpu.SemaphoreType.DMA,
        pltpu.SemaphoreType.DMA,
    ],
)
def _lfm_sc(x_hbm, table_hbm, out_hbm, idx_v, hi_v, buf_a, buf_b, out_v,
            sem_a, sem_b):
    _lfm_body(x_hbm, table_hbm, out_hbm, idx_v, hi_v, buf_a, buf_b, out_v,
              sem_a, sem_b)


def kernel(x, table):
    x2 = x.astype(jnp.int32).reshape(NW * NCHUNK, ICHUNK)
    tt = table.T
    tail8 = table[NCOL * ICHUNK:].reshape(PACK, PACK * D)
    t128 = _retile_sc(tt, tail8)
    out = _lfm_sc(x2, t128)
    return out.reshape(B, 1)


# speedup vs baseline: 2.2270x; 2.1687x over previous
import functools

import jax
import jax.numpy as jnp
from jax import lax
from jax.experimental import pallas as pl
from jax.experimental.pallas import tpu as pltpu
from jax.experimental.pallas import tpu_sc as plsc

B = 16384
D = 16
FEAT = 1000000
PACK = 8
ROW128 = FEAT // PACK
NC = 2
NS = 16
L = 16
NW = NC * NS
BPW = B // NW
IPW = 2 * BPW
ICHUNK = 128
NCHUNK = IPW // ICHUNK
EPC = ICHUNK // 2
GPC = EPC // L

NCOL = FEAT // ICHUNK
CPW = NCOL // NW
NEXTRA = NCOL - CPW * NW
MAINL = NCOL * L


def _retile_body(tt_hbm, tail_hbm, w2_hbm, buf_a, buf_b, tb_a, tb_b,
                 sin_a, sin_b, sout_a, sout_b):
    wid = lax.axis_index("s") * NC + lax.axis_index("c")
    base = wid * CPW
    lanes = lax.iota(jnp.int32, L)

    bufs = [buf_a, buf_b]
    tbs = [tb_a, tb_b]
    sins = [sin_a, sin_b]
    souts = [sout_a, sout_b]

    def start_in(c, p):
        pltpu.make_async_copy(
            tt_hbm.at[:, pl.ds(c * ICHUNK, ICHUNK)], bufs[p], sins[p]
        ).start()

    def wait_in(p):
        pltpu.make_async_copy(
            tt_hbm.at[:, pl.ds(0, ICHUNK)], bufs[p], sins[p]
        ).wait()

    def start_out(c, p):
        pltpu.make_async_copy(
            tbs[p], w2_hbm.at[pl.ds(c * L, L), :], souts[p]
        ).start()

    def wait_out(p):
        pltpu.make_async_copy(
            tbs[p], w2_hbm.at[pl.ds(0, L), :], souts[p]
        ).wait()

    def transpose_col(p):
        buf, tb = bufs[p], tbs[p]
        for s in range(L):
            perm = (lanes + s) & (L - 1)
            colout = ((perm & (PACK - 1)) << 4) + lanes
            rowbase = lax.shift_right_logical(perm, 3)
            for q in range(ICHUNK // L):
                c = perm + L * q
                row = rowbase + 2 * q
                v = plsc.load_gather(buf, [lanes, c])
                plsc.store_scatter(tb, [row, colout], v)

    start_in(base, 0)

    def body(j, _):
        c0 = base + 2 * j
        start_in(c0 + 1, 1)
        wait_in(0)
        transpose_col(0)

        @pl.when(j > 0)
        def _():
            wait_out(0)

        start_out(c0, 0)

        @pl.when(j < CPW // 2 - 1)
        def _():
            start_in(c0 + 2, 0)

        wait_in(1)
        transpose_col(1)

        @pl.when(j > 0)
        def _():
            wait_out(1)

        start_out(c0 + 1, 1)
        return 0

    lax.fori_loop(0, CPW // 2, body, 0)
    wait_out(0)
    wait_out(1)

    for e in range(NEXTRA):
        @pl.when(wid == e)
        def _(e=e):
            c = NW * CPW + e
            pltpu.sync_copy(tt_hbm.at[:, pl.ds(c * ICHUNK, ICHUNK)], buf_a)
            transpose_col(0)
            pltpu.sync_copy(tb_a, w2_hbm.at[pl.ds(c * L, L), :])

    @pl.when(wid == NW - 1)
    def _():
        pltpu.sync_copy(tail_hbm, tb_a.at[pl.ds(0, PACK)])
        pltpu.sync_copy(tb_a.at[pl.ds(0, PACK)], w2_hbm.at[pl.ds(MAINL, PACK), :])


@functools.partial(
    pl.kernel,
    out_type=jax.ShapeDtypeStruct((ROW128, PACK * D), jnp.float32),
    mesh=plsc.VectorSubcoreMesh(core_axis_name="c", subcore_axis_name="s"),
    compiler_params=pltpu.CompilerParams(needs_layout_passes=False),
    scratch_types=[
        pltpu.VMEM((D, ICHUNK), jnp.float32),
        pltpu.VMEM((D, ICHUNK), jnp.float32),
        pltpu.VMEM((L, PACK * D), jnp.float32),
        pltpu.VMEM((L, PACK * D), jnp.float32),
        pltpu.SemaphoreType.DMA,
        pltpu.SemaphoreType.DMA,
        pltpu.SemaphoreType.DMA,
        pltpu.SemaphoreType.DMA,
    ],
)
def _retile_sc(tt_hbm, tail_hbm, w2_hbm, buf_a, buf_b, tb_a, tb_b,
               sin_a, sin_b, sout_a, sout_b):
    _retile_body(tt_hbm, tail_hbm, w2_hbm, buf_a, buf_b, tb_a, tb_b,
                 sin_a, sin_b, sout_a, sout_b)


def _lfm_body(x_hbm, table_hbm, out_hbm, idx_v, hi_v, buf_a, buf_b, out_v,
              sem_a, sem_b):
    wid = lax.axis_index("s") * NC + lax.axis_index("c")

    pltpu.sync_copy(x_hbm.at[pl.ds(wid * NCHUNK, NCHUNK)], idx_v)
    for t in range(NCHUNK):
        for c in range(ICHUNK // L):
            hi_v[t, pl.ds(c * L, L)] = lax.shift_right_logical(
                idx_v[t, pl.ds(c * L, L)], 3
            )

    bufs = [buf_a, buf_b]
    sems = [sem_a, sem_b]
    lanes = lax.iota(jnp.int32, L)

    def fire(j):
        return pltpu.async_copy(
            table_hbm.at[hi_v.at[j]], bufs[j % 2], sems[j % 2]
        )

    def compute(j):
        buf = bufs[j % 2]
        jvec = jnp.full((L,), j, jnp.int32)
        for g in range(GPC):
            k0 = 2 * (g * L) + 2 * lanes
            k1 = k0 + 1
            i0 = plsc.load_gather(idx_v, [jvec, k0])
            i1 = plsc.load_gather(idx_v, [jvec, k1])
            c0 = (i0 & 7) * D
            c1 = (i1 & 7) * D
            acc = jnp.zeros((L,), jnp.float32)
            for d in range(D):
                a = plsc.load_gather(buf, [k0, c0 + d])
                b = plsc.load_gather(buf, [k1, c1 + d])
                acc = acc + a * b
            out_v[pl.ds(j * EPC + g * L, L)] = 1.0 / (1.0 + jnp.exp(-acc))

    copies = [fire(0), fire(1)]
    for j in range(NCHUNK):
        copies[j].wait()
        compute(j)
        if j + 2 < NCHUNK:
            copies.append(fire(j + 2))

    pltpu.sync_copy(out_v, out_hbm.at[pl.ds(wid * BPW, BPW)])


@functools.partial(
    pl.kernel,
    out_type=jax.ShapeDtypeStruct((B,), jnp.float32),
    mesh=plsc.VectorSubcoreMesh(core_axis_name="c", subcore_axis_name="s"),
    compiler_params=pltpu.CompilerParams(needs_layout_passes=False),
    scratch_types=[
        pltpu.VMEM((NCHUNK, ICHUNK), jnp.int32),
        pltpu.VMEM((NCHUNK, ICHUNK), jnp.int32),
        pltpu.VMEM((ICHUNK, PACK * D), jnp.float32),
        pltpu.VMEM((ICHUNK, PACK * D), jnp.float32),
        pltpu.VMEM((BPW,), jnp.float32),
        pltpu.SemaphoreType.DMA,
        pltpu.SemaphoreType.DMA,
    ],
)
def _lfm_sc(x_hbm, table_hbm, out_hbm, idx_v, hi_v, buf_a, buf_b, out_v,
            sem_a, sem_b):
    _lfm_body(x_hbm, table_hbm, out_hbm, idx_v, hi_v, buf_a, buf_b, out_v,
              sem_a, sem_b)


def kernel(x, table):
    x2 = x.astype(jnp.int32).reshape(NW * NCHUNK, ICHUNK)
    tt = table.T
    tail8 = table[NCOL * ICHUNK:].reshape(PACK, PACK * D)
    t128 = _retile_sc(tt, tail8)
    out = _lfm_sc(x2, t128)
    return out.reshape(B, 1)


# speedup vs baseline: 2.2767x; 1.0223x over previous
import functools

import jax
import jax.numpy as jnp
from jax import lax
from jax.experimental import pallas as pl
from jax.experimental.pallas import tpu as pltpu
from jax.experimental.pallas import tpu_sc as plsc

B = 16384
D = 16
FEAT = 1000000
PACK = 8
ROW128 = FEAT // PACK
NC = 2
NS = 16
L = 16
NW = NC * NS
BPW = B // NW
IPW = 2 * BPW
ICHUNK = 128
NCHUNK = IPW // ICHUNK
EPC = ICHUNK // 2
GPC = EPC // L

NCOL = FEAT // ICHUNK
CPW = NCOL // NW
NEXTRA = NCOL - CPW * NW
MAINL = NCOL * L


def _retile_body(tt_hbm, tail_hbm, w2_hbm, buf_a, buf_b, tb_a, tb_b,
                 sin_a, sin_b, sout_a, sout_b):
    wid = lax.axis_index("s") * NC + lax.axis_index("c")
    base = wid * CPW
    lanes = lax.iota(jnp.int32, L)

    bufs = [buf_a, buf_b]
    tbs = [tb_a, tb_b]
    sins = [sin_a, sin_b]
    souts = [sout_a, sout_b]

    def start_in(c, p):
        pltpu.make_async_copy(
            tt_hbm.at[:, pl.ds(c * ICHUNK, ICHUNK)], bufs[p], sins[p]
        ).start()

    def wait_in(p):
        pltpu.make_async_copy(
            tt_hbm.at[:, pl.ds(0, ICHUNK)], bufs[p], sins[p]
        ).wait()

    def start_out(c, p):
        pltpu.make_async_copy(
            tbs[p], w2_hbm.at[pl.ds(c * L, L), :], souts[p]
        ).start()

    def wait_out(p):
        pltpu.make_async_copy(
            tbs[p], w2_hbm.at[pl.ds(0, L), :], souts[p]
        ).wait()

    def transpose_col(p):
        buf, tb = bufs[p], tbs[p]
        colouts = [((lanes & (PACK - 1)) << 4) + ((d + lanes) & (L - 1))
                   for d in range(D)]
        for k in range(ICHUNK // L):
            row = 2 * k + lax.shift_right_logical(lanes, 3)
            for d in range(D):
                v = buf[d, pl.ds(L * k, L)]
                plsc.store_scatter(tb, [row, colouts[d]], v)

    start_in(base, 0)

    def body(j, _):
        c0 = base + 2 * j
        start_in(c0 + 1, 1)
        wait_in(0)
        transpose_col(0)

        @pl.when(j > 0)
        def _():
            wait_out(0)

        start_out(c0, 0)

        @pl.when(j < CPW // 2 - 1)
        def _():
            start_in(c0 + 2, 0)

        wait_in(1)
        transpose_col(1)

        @pl.when(j > 0)
        def _():
            wait_out(1)

        start_out(c0 + 1, 1)
        return 0

    lax.fori_loop(0, CPW // 2, body, 0)
    wait_out(0)
    wait_out(1)

    for e in range(NEXTRA):
        @pl.when(wid == e)
        def _(e=e):
            c = NW * CPW + e
            pltpu.sync_copy(tt_hbm.at[:, pl.ds(c * ICHUNK, ICHUNK)], buf_a)
            transpose_col(0)
            pltpu.sync_copy(tb_a, w2_hbm.at[pl.ds(c * L, L), :])

    @pl.when(wid == NW - 1)
    def _():
        pltpu.sync_copy(tail_hbm, tb_a.at[pl.ds(0, PACK)])
        pltpu.sync_copy(tb_a.at[pl.ds(0, PACK)], w2_hbm.at[pl.ds(MAINL, PACK), :])


@functools.partial(
    pl.kernel,
    out_type=jax.ShapeDtypeStruct((ROW128, PACK * D), jnp.float32),
    mesh=plsc.VectorSubcoreMesh(core_axis_name="c", subcore_axis_name="s"),
    compiler_params=pltpu.CompilerParams(needs_layout_passes=False),
    scratch_types=[
        pltpu.VMEM((D, ICHUNK), jnp.float32),
        pltpu.VMEM((D, ICHUNK), jnp.float32),
        pltpu.VMEM((L, PACK * D), jnp.float32),
        pltpu.VMEM((L, PACK * D), jnp.float32),
        pltpu.SemaphoreType.DMA,
        pltpu.SemaphoreType.DMA,
        pltpu.SemaphoreType.DMA,
        pltpu.SemaphoreType.DMA,
    ],
)
def _retile_sc(tt_hbm, tail_hbm, w2_hbm, buf_a, buf_b, tb_a, tb_b,
               sin_a, sin_b, sout_a, sout_b):
    _retile_body(tt_hbm, tail_hbm, w2_hbm, buf_a, buf_b, tb_a, tb_b,
                 sin_a, sin_b, sout_a, sout_b)


def _lfm_body(x_hbm, table_hbm, out_hbm, idx_v, hi_v, buf_a, buf_b, out_v,
              sem_a, sem_b):
    wid = lax.axis_index("s") * NC + lax.axis_index("c")

    pltpu.sync_copy(x_hbm.at[pl.ds(wid * NCHUNK, NCHUNK)], idx_v)
    for t in range(NCHUNK):
        for c in range(ICHUNK // L):
            hi_v[t, pl.ds(c * L, L)] = lax.shift_right_logical(
                idx_v[t, pl.ds(c * L, L)], 3
            )

    bufs = [buf_a, buf_b]
    sems = [sem_a, sem_b]
    lanes = lax.iota(jnp.int32, L)

    def fire(j):
        return pltpu.async_copy(
            table_hbm.at[hi_v.at[j]], bufs[j % 2], sems[j % 2]
        )

    ks_tab = [(lanes + s) & (L - 1) for s in range(D)]

    def compute(j):
        buf = bufs[j % 2]
        jvec = jnp.full((L,), j, jnp.int32)
        for g in range(GPC):
            k0 = 2 * (g * L) + 2 * lanes
            k1 = k0 + 1
            i0 = plsc.load_gather(idx_v, [jvec, k0])
            i1 = plsc.load_gather(idx_v, [jvec, k1])
            c0 = (i0 & 7) * D
            c1 = (i1 & 7) * D
            m = (i1 - i0) & (L - 1)
            acc = jnp.zeros((L,), jnp.float32)
            for s in range(D):
                a = plsc.load_gather(buf, [k0, c0 + ks_tab[s]])
                b = plsc.load_gather(buf, [k1, c1 + ((ks_tab[s] + m) & (L - 1))])
                acc = acc + a * b
            out_v[pl.ds(j * EPC + g * L, L)] = 1.0 / (1.0 + jnp.exp(-acc))

    copies = [fire(0), fire(1)]
    for j in range(NCHUNK):
        copies[j].wait()
        compute(j)
        if j + 2 < NCHUNK:
            copies.append(fire(j + 2))

    pltpu.sync_copy(out_v, out_hbm.at[pl.ds(wid * BPW, BPW)])


@functools.partial(
    pl.kernel,
    out_type=jax.ShapeDtypeStruct((B,), jnp.float32),
    mesh=plsc.VectorSubcoreMesh(core_axis_name="c", subcore_axis_name="s"),
    compiler_params=pltpu.CompilerParams(needs_layout_passes=False),
    scratch_types=[
        pltpu.VMEM((NCHUNK, ICHUNK), jnp.int32),
        pltpu.VMEM((NCHUNK, ICHUNK), jnp.int32),
        pltpu.VMEM((ICHUNK, PACK * D), jnp.float32),
        pltpu.VMEM((ICHUNK, PACK * D), jnp.float32),
        pltpu.VMEM((BPW,), jnp.float32),
        pltpu.SemaphoreType.DMA,
        pltpu.SemaphoreType.DMA,
    ],
)
def _lfm_sc(x_hbm, table_hbm, out_hbm, idx_v, hi_v, buf_a, buf_b, out_v,
            sem_a, sem_b):
    _lfm_body(x_hbm, table_hbm, out_hbm, idx_v, hi_v, buf_a, buf_b, out_v,
              sem_a, sem_b)


def kernel(x, table):
    x2 = x.astype(jnp.int32).reshape(NW * NCHUNK, ICHUNK)
    tt = table.T
    tailm = table[NCOL * ICHUNK:]
    j64 = jnp.arange(64, dtype=jnp.int32)[:, None]
    p16 = jnp.arange(D, dtype=jnp.int32)[None, :]
    tail8 = jnp.take_along_axis(tailm, (p16 - j64) % D, axis=1)
    tail8 = tail8.reshape(PACK, PACK * D)
    t128 = _retile_sc(tt, tail8)
    out = _lfm_sc(x2, t128)
    return out.reshape(B, 1)
